# baseline (speedup 1.0000x reference)
import functools

import jax
import jax.numpy as jnp
from jax import lax
from jax.experimental import pallas as pl
from jax.experimental.pallas import tpu as pltpu
from jax.experimental.pallas import tpu_sc as plsc

N = 50000
E = 800000
TW = 25
HID = 64
DC = 8
L_PDE = 16.0
TMAX = 4.0
DT = 0.2

NC, NS, LANES = 2, 16, 16
NW = NC * NS
N_PAD = 51200
RPT = N_PAD // NS

E_PAD = 819200
EPT_A = E_PAD // NW
CH_A = 1280
NCH_A = EPT_A // CH_A
GRP_A = CH_A // LANES

EPT_C = E_PAD // NS
CH_C = 512
NCH_C = EPT_C // CH_C
SUB = 128
NSUB = CH_C // SUB
HH = HID // 2

R_B = 2048
N_BLOCKS = N_PAD // R_B

_MESH = plsc.VectorSubcoreMesh(
    core_axis_name="c", subcore_axis_name="s", num_cores=NC, num_subcores=NS)


def _deg_body(px_hbm, row_hbm, col_hbm, degpart_hbm, ew_hbm,
              px_v, rowb, colb, ewb, degp):
    c = lax.axis_index("c")
    s = lax.axis_index("s")
    wid = c * NS + s
    pltpu.sync_copy(px_hbm, px_v)
    zeros = jnp.zeros((LANES,), jnp.float32)

    def _zero(i, _):
        degp[pl.ds(i * LANES, LANES)] = zeros
        return ()
    lax.fori_loop(0, N_PAD // LANES, _zero, (), unroll=8)

    base_e = wid * EPT_A

    def _chunk(ch, _):
        off = base_e + ch * CH_A
        pltpu.sync_copy(row_hbm.at[pl.ds(off, CH_A)], rowb)
        pltpu.sync_copy(col_hbm.at[pl.ds(off, CH_A)], colb)

        def _grp(g, _):
            idr = rowb[pl.ds(g * LANES, LANES)]
            idc = colb[pl.ds(g * LANES, LANES)]
            a = plsc.load_gather(px_v, [idr])
            b = plsc.load_gather(px_v, [idc])
            ew = jnp.abs(a - b)
            ewb[pl.ds(g * LANES, LANES)] = ew
            plsc.addupdate_scatter(degp, [idc], ew)
            return ()
        lax.fori_loop(0, GRP_A, _grp, ())
        pltpu.sync_copy(ewb, ew_hbm.at[pl.ds(off, CH_A)])
        return ()
    lax.fori_loop(0, NCH_A, _chunk, ())
    pltpu.sync_copy(degp, degpart_hbm.at[wid])


_deg_call = pl.kernel(
    _deg_body,
    out_type=(jax.ShapeDtypeStruct((NW, N_PAD), jnp.float32),
              jax.ShapeDtypeStruct((E_PAD,), jnp.float32)),
    mesh=_MESH,
    scratch_types=[
        pltpu.VMEM((N_PAD,), jnp.float32),
        pltpu.VMEM((CH_A,), jnp.int32),
        pltpu.VMEM((CH_A,), jnp.int32),
        pltpu.VMEM((CH_A,), jnp.float32),
        pltpu.VMEM((N_PAD,), jnp.float32),
    ],
    compiler_params=pltpu.CompilerParams(needs_layout_passes=False,
                                         use_tc_tiling_on_sc=False),
)


def _msg_body(rccat_hbm, ew_hbm, ycat_hbm, acc_hbm,
              idxb, ewb, rows_bf, rows_v, acc_sh, gsem, ssem, isem):
    c = lax.axis_index("c")
    s = lax.axis_index("s")
    zeros = jnp.zeros((LANES,), jnp.float32)

    def _zero(i, _):
        rows_v[i, pl.ds(0, LANES)] = zeros
        rows_v[i, pl.ds(LANES, LANES)] = zeros
        return ()
    lax.fori_loop(0, CH_C, _zero, (), unroll=8)

    r0 = s * RPT
    for k in range(RPT // CH_C):
        pltpu.sync_copy(rows_v, acc_sh.at[pl.ds(r0 + k * CH_C, CH_C)])
    if RPT % CH_C:
        pltpu.sync_copy(rows_v.at[pl.ds(0, RPT % CH_C)],
                        acc_sh.at[pl.ds(r0 + (RPT // CH_C) * CH_C, RPT % CH_C)])
    plsc.subcore_barrier()

    gch0 = s * NCH_C
    ebase = s * EPT_C

    def _fire_idx(ch, p):
        pltpu.make_async_copy(rccat_hbm.at[c, gch0 + ch],
                              idxb.at[p], isem.at[p]).start()
        pltpu.make_async_copy(ew_hbm.at[pl.ds(ebase + ch * CH_C, CH_C)],
                              ewb.at[p], isem.at[p]).start()

    def _wait_idx(p):
        pltpu.make_async_copy(rccat_hbm.at[c, 0], idxb.at[p],
                              isem.at[p]).wait()
        pltpu.make_async_copy(ew_hbm.at[pl.ds(0, CH_C)], ewb.at[p],
                              isem.at[p]).wait()

    def _fire_gather(p, j):
        pltpu.make_async_copy(ycat_hbm.at[idxb.at[p, j]],
                              rows_bf.at[pl.ds(j * SUB, SUB)],
                              gsem.at[j]).start()

    def _wait_gather(p, j):
        pltpu.make_async_copy(ycat_hbm.at[idxb.at[p, j]],
                              rows_bf.at[pl.ds(j * SUB, SUB)],
                              gsem.at[j]).wait()

    def _fire_scatter(p, j):
        pltpu.make_async_copy(rows_v.at[pl.ds(j * SUB, SUB)],
                              acc_sh.at[idxb.at[p, NSUB + j]],
                              ssem.at[j]).start(add=True)

    def _wait_scatter(p, j):
        pltpu.make_async_copy(rows_v.at[pl.ds(j * SUB, SUB)],
                              acc_sh.at[idxb.at[p, NSUB + j]],
                              ssem.at[j]).wait()

    def _process(p):
        for j in range(NSUB):
            _wait_gather(p, j)

            def _scale(g, _):
                off = j * SUB + g * LANES
                ewv = ewb[p, pl.ds(off, LANES)]
                himask = jnp.full((LANES,), -65536, jnp.int32)
                for b in range(LANES):
                    e = off + b
                    w = ewv[b]
                    wv = plsc.bitcast(rows_bf[e, pl.ds(0, 2 * LANES)],
                                      jnp.int32)
                    lo = plsc.bitcast(lax.shift_left(wv, 16), jnp.float32)
                    hi = plsc.bitcast(lax.bitwise_and(wv, himask),
                                      jnp.float32)
                    rows_v[e, pl.ds(0, LANES)] = lo * w
                    rows_v[e, pl.ds(LANES, LANES)] = hi * w
                return ()
            lax.fori_loop(0, SUB // LANES, _scale, ())
            _fire_scatter(p, j)

    _fire_idx(0, 0)
    _wait_idx(0)
    for j in range(NSUB):
        _fire_gather(0, j)
    _fire_idx(1, 1)

    def _pair(i, _):
        for p in range(2):
            ch = 2 * i + p
            _process(p)
            _wait_idx(1 - p)
            for j in range(NSUB):
                _wait_scatter(p, j)
                _fire_gather(1 - p, j)
            _fire_idx(ch + 2, p)
        return ()
    lax.fori_loop(0, NCH_C // 2 - 1, _pair, ())

    _process(0)
    _wait_idx(1)
    for j in range(NSUB):
        _wait_scatter(0, j)
        _fire_gather(1, j)
    _process(1)
    for j in range(NSUB):
        _wait_scatter(1, j)

    plsc.subcore_barrier()
    pltpu.sync_copy(acc_sh.at[pl.ds(r0, RPT)],
                    acc_hbm.at[c, pl.ds(r0, RPT)])


_msg_call = pl.kernel(
    _msg_body,
    out_type=jax.ShapeDtypeStruct((NC, N_PAD, HH), jnp.float32),
    mesh=_MESH,
    scratch_types=[
        pltpu.VMEM((2, 2 * NSUB, SUB), jnp.int32),
        pltpu.VMEM((2, CH_C), jnp.float32),
        pltpu.VMEM((CH_C, HH), jnp.bfloat16),
        pltpu.VMEM((CH_C, HH), jnp.float32),
        pltpu.VMEM_SHARED((N_PAD, HH), jnp.float32),
        pltpu.SemaphoreType.DMA((NSUB,)),
        pltpu.SemaphoreType.DMA((NSUB,)),
        pltpu.SemaphoreType.DMA((2,)),
    ],
    compiler_params=pltpu.CompilerParams(needs_layout_passes=False,
                                         use_tc_tiling_on_sc=False),
)


def _swish(v):
    return v * lax.logistic(v)


def _encprep_body(ni_ref, W1_ref, b1_ref, W2_ref, b2_ref, Wg_ref, bg_ref,
                  degp_ref, y2_ref, base_ref, dinv_ref):
    ni = ni_ref[...]
    h = _swish(jnp.dot(ni, W1_ref[...],
                       preferred_element_type=jnp.float32) + b1_ref[...])
    h0 = _swish(jnp.dot(h, W2_ref[...],
                        preferred_element_type=jnp.float32) + b2_ref[...])
    xw = jnp.dot(h0, Wg_ref[...], preferred_element_type=jnp.float32)
    deg = jnp.sum(degp_ref[...], axis=0) + 1.0
    dinv = lax.rsqrt(deg)
    y = xw * dinv[:, None]
    yb = y.astype(jnp.bfloat16)
    y2_ref[0] = yb[:, :HH]
    y2_ref[1] = yb[:, HH:]
    base_ref[...] = h0 + DT * (y * dinv[:, None] + bg_ref[...])
    dinv_ref[...] = dinv[:, None]


def _dec_body(base_ref, acc0_ref, acc1_ref, dinv_ref, Wc1_ref, cb1_ref,
              Wc2_ref, cb2_ref, Wo_ref, bo_ref, out_ref):
    dinv = dinv_ref[...]

    def _unperm(a):
        r = a.shape[0]
        return jnp.stack([a[:, :HH // 2], a[:, HH // 2:]], axis=-1).reshape(
            r, HH)

    sfull = jnp.concatenate([_unperm(acc0_ref[...]),
                             _unperm(acc1_ref[...])], axis=1) * dinv
    Hm = base_ref[...] + DT * sfull
    z1 = _swish(jnp.dot(Hm, Wc1_ref[...],
                        preferred_element_type=jnp.float32) + cb1_ref[...])
    z2 = _swish(jnp.dot(z1, Wc2_ref[...],
                        preferred_element_type=jnp.float32) + cb2_ref[...])
    out_ref[...] = jnp.dot(z2, Wo_ref[...],
                           preferred_element_type=jnp.float32) + bo_ref[...]


def _full(shape):
    return pl.BlockSpec(shape, lambda i: (0,) * len(shape))


def _rows(width):
    return pl.BlockSpec((R_B, width), lambda i: (i, 0))


_encprep_call = pl.pallas_call(
    _encprep_body,
    grid=(N_BLOCKS,),
    in_specs=[_rows(TW + 2), _full((TW + 2, HID)), _full((1, HID)),
              _full((HID, HID)), _full((1, HID)), _full((HID, HID)),
              _full((1, HID)), pl.BlockSpec((NW, R_B), lambda i: (0, i))],
    out_specs=[pl.BlockSpec((2, R_B, HH), lambda i: (0, i, 0)),
               _rows(HID), _rows(1)],
    out_shape=[jax.ShapeDtypeStruct((2, N_PAD, HH), jnp.bfloat16),
               jax.ShapeDtypeStruct((N_PAD, HID), jnp.float32),
               jax.ShapeDtypeStruct((N_PAD, 1), jnp.float32)],
)

_dec_call = pl.pallas_call(
    _dec_body,
    grid=(N_BLOCKS,),
    in_specs=[_rows(HID), _rows(HH), _rows(HH), _rows(1),
              _full((HID, DC * 17)), _full((1, DC * 17)),
              _full((DC * 17, 4)), _full((1, 1)),
              _full((4, TW)), _full((1, TW))],
    out_specs=_rows(TW),
    out_shape=jax.ShapeDtypeStruct((N_PAD, TW), jnp.float32),
)


def kernel(x, pos, edge_index, batch, W1, b1, W2, b2, Wg, bg,
           cw1, cb1, cw2, cb2, Wo, bo):
    px = pos[:, 1] * jnp.float32(1.0 / L_PDE)
    pt = pos[:, 0] * jnp.float32(1.0 / TMAX)
    ni = jnp.concatenate([x, px[:, None], pt[:, None]], axis=1)

    px_pad = jnp.pad(px, (0, N_PAD - N))
    row = edge_index[0]
    col = edge_index[1]
    rowp = jnp.pad(row, (0, E_PAD - E))
    colp = jnp.pad(col, (0, E_PAD - E))

    degpart, ew = _deg_call(px_pad, rowp, colp)
    y2, base, dinv = _encprep_call(ni, W1, b1[None, :], W2, b2[None, :], Wg,
                                   bg[None, :], degpart)

    ycat = y2.reshape(2 * N_PAD, HH)
    row3 = rowp.reshape(E_PAD // CH_C, NSUB, SUB)
    col3 = colp.reshape(E_PAD // CH_C, NSUB, SUB)
    rc = jnp.concatenate([row3, col3], axis=1)
    rccat = jnp.stack([rc, rc.at[:, :NSUB].add(N_PAD)])
    acc = _msg_call(rccat, ew, ycat)

    cw1t = jnp.transpose(cw1[:, 0, :])
    Wc1 = jnp.zeros((HID, DC, 17), jnp.float32)
    for l in range(17):
        Wc1 = Wc1.at[3 * l:3 * l + 16, :, l].set(cw1t)
    Wc1 = Wc1.reshape(HID, DC * 17)
    cb1e = jnp.repeat(cb1, 17)[None, :]
    Wc2 = jnp.zeros((DC, 17, 4), jnp.float32)
    for l2 in range(4):
        Wc2 = Wc2.at[:, l2:l2 + 14, l2].set(cw2[0])
    Wc2 = Wc2.reshape(DC * 17, 4)

    out = _dec_call(base, acc[0], acc[1], dinv,
                    Wc1, cb1e, Wc2, cb2[None, :], Wo, bo[None, :])
    return out[:N]


# speedup vs baseline: 1.9220x; 1.9220x over previous
import functools

import jax
import jax.numpy as jnp
from jax import lax
from jax.experimental import pallas as pl
from jax.experimental.pallas import tpu as pltpu
from jax.experimental.pallas import tpu_sc as plsc

N = 50000
E = 800000
TW = 25
HID = 64
DC = 8
L_PDE = 16.0
TMAX = 4.0
DT = 0.2

NC, NS, LANES = 2, 16, 16
NW = NC * NS
N_PAD = 51200
RPT = N_PAD // NS

E_PAD = 819200
EPT_A = E_PAD // NW
CH_A = 1280
NCH_A = EPT_A // CH_A
GRP_A = CH_A // LANES

EPT_C = E_PAD // NS
CH_C = 512
NCH_C = EPT_C // CH_C
SUB = 128
NSUB = CH_C // SUB
HH = HID // 2

R_B = 2048
N_BLOCKS = N_PAD // R_B

_MESH = plsc.VectorSubcoreMesh(
    core_axis_name="c", subcore_axis_name="s", num_cores=NC, num_subcores=NS)


def _deg_body(px_hbm, row_hbm, col_hbm, degpart_hbm, ew_hbm,
              px_v, rowb, colb, ewb, degp):
    c = lax.axis_index("c")
    s = lax.axis_index("s")
    wid = c * NS + s
    pltpu.sync_copy(px_hbm, px_v)
    zeros = jnp.zeros((LANES,), jnp.float32)

    def _zero(i, _):
        degp[pl.ds(i * LANES, LANES)] = zeros
        return ()
    lax.fori_loop(0, N_PAD // LANES, _zero, (), unroll=8)

    base_e = wid * EPT_A

    def _chunk(ch, _):
        off = base_e + ch * CH_A
        pltpu.sync_copy(row_hbm.at[pl.ds(off, CH_A)], rowb)
        pltpu.sync_copy(col_hbm.at[pl.ds(off, CH_A)], colb)

        def _grp(g, _):
            idr = rowb[pl.ds(g * LANES, LANES)]
            idc = colb[pl.ds(g * LANES, LANES)]
            a = plsc.load_gather(px_v, [idr])
            b = plsc.load_gather(px_v, [idc])
            ew = jnp.abs(a - b)
            ewb[pl.ds(g * LANES, LANES)] = ew
            plsc.addupdate_scatter(degp, [idc], ew)
            return ()
        lax.fori_loop(0, GRP_A, _grp, ())
        pltpu.sync_copy(ewb, ew_hbm.at[pl.ds(off, CH_A)])
        return ()
    lax.fori_loop(0, NCH_A, _chunk, ())
    pltpu.sync_copy(degp, degpart_hbm.at[wid])


_deg_call = pl.kernel(
    _deg_body,
    out_type=(jax.ShapeDtypeStruct((NW, N_PAD), jnp.float32),
              jax.ShapeDtypeStruct((E_PAD,), jnp.float32)),
    mesh=_MESH,
    scratch_types=[
        pltpu.VMEM((N_PAD,), jnp.float32),
        pltpu.VMEM((CH_A,), jnp.int32),
        pltpu.VMEM((CH_A,), jnp.int32),
        pltpu.VMEM((CH_A,), jnp.float32),
        pltpu.VMEM((N_PAD,), jnp.float32),
    ],
    compiler_params=pltpu.CompilerParams(needs_layout_passes=False,
                                         use_tc_tiling_on_sc=False),
)


def _msg_body(rccat_hbm, ew_hbm, ycat_hbm, acc_hbm,
              idxb, ewb, rows_v, acc_sh, gsem, ssem, isem):
    c = lax.axis_index("c")
    s = lax.axis_index("s")
    zeros = jnp.zeros((LANES,), jnp.float32)

    def _zero(i, _):
        rows_v[i, pl.ds(0, LANES)] = zeros
        rows_v[i, pl.ds(LANES, LANES)] = zeros
        return ()
    lax.fori_loop(0, CH_C, _zero, (), unroll=8)

    r0 = s * RPT
    for k in range(RPT // CH_C):
        pltpu.sync_copy(rows_v, acc_sh.at[pl.ds(r0 + k * CH_C, CH_C)])
    if RPT % CH_C:
        pltpu.sync_copy(rows_v.at[pl.ds(0, RPT % CH_C)],
                        acc_sh.at[pl.ds(r0 + (RPT // CH_C) * CH_C, RPT % CH_C)])
    plsc.subcore_barrier()

    gch0 = s * NCH_C
    ebase = s * EPT_C

    def _fire_idx(ch, p):
        pltpu.make_async_copy(rccat_hbm.at[c, gch0 + ch],
                              idxb.at[p], isem.at[p]).start()
        pltpu.make_async_copy(ew_hbm.at[pl.ds(ebase + ch * CH_C, CH_C)],
                              ewb.at[p], isem.at[p]).start()

    def _wait_idx(p):
        pltpu.make_async_copy(rccat_hbm.at[c, 0], idxb.at[p],
                              isem.at[p]).wait()
        pltpu.make_async_copy(ew_hbm.at[pl.ds(0, CH_C)], ewb.at[p],
                              isem.at[p]).wait()

    def _fire_gather(p, j):
        pltpu.make_async_copy(ycat_hbm.at[idxb.at[p, j]],
                              rows_v.at[pl.ds(j * SUB, SUB)],
                              gsem.at[j]).start()

    def _wait_gather(p, j):
        pltpu.make_async_copy(ycat_hbm.at[idxb.at[p, j]],
                              rows_v.at[pl.ds(j * SUB, SUB)],
                              gsem.at[j]).wait()

    def _fire_scatter(p, j):
        pltpu.make_async_copy(rows_v.at[pl.ds(j * SUB, SUB)],
                              acc_sh.at[idxb.at[p, NSUB + j]],
                              ssem.at[j]).start(add=True)

    def _wait_scatter(p, j):
        pltpu.make_async_copy(rows_v.at[pl.ds(j * SUB, SUB)],
                              acc_sh.at[idxb.at[p, NSUB + j]],
                              ssem.at[j]).wait()

    def _process(p):
        for j in range(NSUB):
            _wait_gather(p, j)

            def _scale(g, _):
                off = j * SUB + g * LANES
                ewv = ewb[p, pl.ds(off, LANES)]
                for b in range(LANES):
                    e = off + b
                    w = ewv[b]
                    rows_v[e, pl.ds(0, LANES)] = (
                        rows_v[e, pl.ds(0, LANES)] * w)
                    rows_v[e, pl.ds(LANES, LANES)] = (
                        rows_v[e, pl.ds(LANES, LANES)] * w)
                return ()
            lax.fori_loop(0, SUB // LANES, _scale, ())
            _fire_scatter(p, j)

    _fire_idx(0, 0)
    _wait_idx(0)
    for j in range(NSUB):
        _fire_gather(0, j)
    _fire_idx(1, 1)

    def _pair(i, _):
        for p in range(2):
            ch = 2 * i + p
            _process(p)
            _wait_idx(1 - p)
            for j in range(NSUB):
                _wait_scatter(p, j)
                _fire_gather(1 - p, j)
            _fire_idx(ch + 2, p)
        return ()
    lax.fori_loop(0, NCH_C // 2 - 1, _pair, ())

    _process(0)
    _wait_idx(1)
    for j in range(NSUB):
        _wait_scatter(0, j)
        _fire_gather(1, j)
    _process(1)
    for j in range(NSUB):
        _wait_scatter(1, j)

    plsc.subcore_barrier()
    pltpu.sync_copy(acc_sh.at[pl.ds(r0, RPT)],
                    acc_hbm.at[c, pl.ds(r0, RPT)])


_msg_call = pl.kernel(
    _msg_body,
    out_type=jax.ShapeDtypeStruct((NC, N_PAD, HH), jnp.float32),
    mesh=_MESH,
    scratch_types=[
        pltpu.VMEM((2, 2 * NSUB, SUB), jnp.int32),
        pltpu.VMEM((2, CH_C), jnp.float32),
        pltpu.VMEM((CH_C, HH), jnp.float32),
        pltpu.VMEM_SHARED((N_PAD, HH), jnp.float32),
        pltpu.SemaphoreType.DMA((NSUB,)),
        pltpu.SemaphoreType.DMA((NSUB,)),
        pltpu.SemaphoreType.DMA((2,)),
    ],
    compiler_params=pltpu.CompilerParams(needs_layout_passes=False,
                                         use_tc_tiling_on_sc=False),
)


def _swish(v):
    return v * lax.logistic(v)


def _encprep_body(ni_ref, W1_ref, b1_ref, W2_ref, b2_ref, Wg_ref, bg_ref,
                  degp_ref, y2_ref, base_ref, dinv_ref):
    ni = ni_ref[...]
    h = _swish(jnp.dot(ni, W1_ref[...],
                       preferred_element_type=jnp.float32) + b1_ref[...])
    h0 = _swish(jnp.dot(h, W2_ref[...],
                        preferred_element_type=jnp.float32) + b2_ref[...])
    xw = jnp.dot(h0, Wg_ref[...], preferred_element_type=jnp.float32)
    deg = jnp.sum(degp_ref[...], axis=0) + 1.0
    dinv = lax.rsqrt(deg)
    y = xw * dinv[:, None]
    y2_ref[0] = y[:, :HH]
    y2_ref[1] = y[:, HH:]
    base_ref[...] = h0 + DT * (y * dinv[:, None] + bg_ref[...])
    dinv_ref[...] = dinv[:, None]


def _dec_body(base_ref, acc0_ref, acc1_ref, dinv_ref, Wc1_ref, cb1_ref,
              Wc2_ref, cb2_ref, Wo_ref, bo_ref, out_ref):
    dinv = dinv_ref[...]

    sfull = jnp.concatenate([acc0_ref[...], acc1_ref[...]], axis=1) * dinv
    Hm = base_ref[...] + DT * sfull
    z1 = _swish(jnp.dot(Hm, Wc1_ref[...],
                        preferred_element_type=jnp.float32) + cb1_ref[...])
    z2 = _swish(jnp.dot(z1, Wc2_ref[...],
                        preferred_element_type=jnp.float32) + cb2_ref[...])
    out_ref[...] = jnp.dot(z2, Wo_ref[...],
                           preferred_element_type=jnp.float32) + bo_ref[...]


def _full(shape):
    return pl.BlockSpec(shape, lambda i: (0,) * len(shape))


def _rows(width):
    return pl.BlockSpec((R_B, width), lambda i: (i, 0))


_encprep_call = pl.pallas_call(
    _encprep_body,
    grid=(N_BLOCKS,),
    in_specs=[_rows(TW + 2), _full((TW + 2, HID)), _full((1, HID)),
              _full((HID, HID)), _full((1, HID)), _full((HID, HID)),
              _full((1, HID)), pl.BlockSpec((NW, R_B), lambda i: (0, i))],
    out_specs=[pl.BlockSpec((2, R_B, HH), lambda i: (0, i, 0)),
               _rows(HID), _rows(1)],
    out_shape=[jax.ShapeDtypeStruct((2, N_PAD, HH), jnp.float32),
               jax.ShapeDtypeStruct((N_PAD, HID), jnp.float32),
               jax.ShapeDtypeStruct((N_PAD, 1), jnp.float32)],
)

_dec_call = pl.pallas_call(
    _dec_body,
    grid=(N_BLOCKS,),
    in_specs=[_rows(HID), _rows(HH), _rows(HH), _rows(1),
              _full((HID, DC * 17)), _full((1, DC * 17)),
              _full((DC * 17, 4)), _full((1, 1)),
              _full((4, TW)), _full((1, TW))],
    out_specs=_rows(TW),
    out_shape=jax.ShapeDtypeStruct((N_PAD, TW), jnp.float32),
)


def kernel(x, pos, edge_index, batch, W1, b1, W2, b2, Wg, bg,
           cw1, cb1, cw2, cb2, Wo, bo):
    px = pos[:, 1] * jnp.float32(1.0 / L_PDE)
    pt = pos[:, 0] * jnp.float32(1.0 / TMAX)
    ni = jnp.concatenate([x, px[:, None], pt[:, None]], axis=1)

    px_pad = jnp.pad(px, (0, N_PAD - N))
    row = edge_index[0]
    col = edge_index[1]
    rowp = jnp.pad(row, (0, E_PAD - E))
    colp = jnp.pad(col, (0, E_PAD - E))

    degpart, ew = _deg_call(px_pad, rowp, colp)
    y2, base, dinv = _encprep_call(ni, W1, b1[None, :], W2, b2[None, :], Wg,
                                   bg[None, :], degpart)

    ycat = y2.reshape(2 * N_PAD, HH)
    row3 = rowp.reshape(E_PAD // CH_C, NSUB, SUB)
    col3 = colp.reshape(E_PAD // CH_C, NSUB, SUB)
    rc = jnp.concatenate([row3, col3], axis=1)
    rccat = jnp.stack([rc, rc.at[:, :NSUB].add(N_PAD)])
    acc = _msg_call(rccat, ew, ycat)

    cw1t = jnp.transpose(cw1[:, 0, :])
    Wc1 = jnp.zeros((HID, DC, 17), jnp.float32)
    for l in range(17):
        Wc1 = Wc1.at[3 * l:3 * l + 16, :, l].set(cw1t)
    Wc1 = Wc1.reshape(HID, DC * 17)
    cb1e = jnp.repeat(cb1, 17)[None, :]
    Wc2 = jnp.zeros((DC, 17, 4), jnp.float32)
    for l2 in range(4):
        Wc2 = Wc2.at[:, l2:l2 + 14, l2].set(cw2[0])
    Wc2 = Wc2.reshape(DC * 17, 4)

    out = _dec_call(base, acc[0], acc[1], dinv,
                    Wc1, cb1e, Wc2, cb2[None, :], Wo, bo[None, :])
    return out[:N]
